# baseline (speedup 1.0000x reference)
import functools

import jax
import jax.numpy as jnp
from jax import lax
from jax.experimental import pallas as pl
from jax.experimental.pallas import tpu as pltpu
from jax.experimental.pallas import tpu_sc as plsc

E = 128
H = 512
NL = 3
G = 4 * H
R = 512
NCHUNK = G // R
NBUF = 4


def _sc_gather(idx, table):
    _, e = table.shape
    mesh = plsc.VectorSubcoreMesh(core_axis_name="c", subcore_axis_name="s")

    @functools.partial(
        pl.kernel,
        out_type=jax.ShapeDtypeStruct((1, e), jnp.float32),
        mesh=mesh,
        scratch_types=[
            pltpu.VMEM((1,), jnp.int32),
            pltpu.VMEM((1, e), jnp.float32),
            pltpu.SemaphoreType.DMA,
        ],
    )
    def k(idx_hbm, table_hbm, out_hbm, idx_v, row_v, sem):
        c = lax.axis_index("c")
        s = lax.axis_index("s")

        @pl.when(jnp.logical_and(c == 0, s == 0))
        def _():
            pltpu.sync_copy(idx_hbm, idx_v)
            pltpu.async_copy(table_hbm.at[idx_v], row_v, sem).wait()
            pltpu.sync_copy(row_v, out_hbm)

    return k(idx, table)


def _dot_nt(a, w):
    return lax.dot_general(
        a, w,
        dimension_numbers=(((1,), (1,)), ((), ())),
        preferred_element_type=jnp.float32,
        precision=lax.Precision.HIGHEST,
    )


def _lstm_body(emb_ref, h6_ref, c6_ref, bih_ref, bhh_ref, *rest):
    w_refs = rest[:12]
    out_h_ref = rest[12]
    out_c_ref = rest[13]
    bufs = rest[14:14 + NBUF]
    sems = rest[14 + NBUF:14 + 2 * NBUF]

    tasks = []
    for l in range(NL):
        ind = E if l == 0 else 2 * H
        for d in range(2):
            m = 2 * (2 * l + d)
            for j in range(NCHUNK):
                tasks.append((m, j * R, ind))
            for j in range(NCHUNK):
                tasks.append((m + 1, j * R, H))
    nt = len(tasks)

    def copy(t):
        m, r0, cc = tasks[t]
        return pltpu.make_async_copy(
            w_refs[m].at[pl.ds(r0, R), pl.ds(0, cc)],
            bufs[t % NBUF].at[:, pl.ds(0, cc)],
            sems[t % NBUF],
        )

    for t in range(min(NBUF, nt)):
        copy(t).start()

    t = 0
    x = emb_ref[...]
    for l in range(NL):
        ind = E if l == 0 else 2 * H
        hs = []
        for d in range(2):
            idx = 2 * l + d
            h_prev = h6_ref[pl.ds(idx, 1), :]
            c_prev = c6_ref[pl.ds(idx, 1), :]
            g_parts = []
            for j in range(NCHUNK):
                copy(t).wait()
                w = bufs[t % NBUF][:, pl.ds(0, ind)]
                g_parts.append(_dot_nt(x, w))
                if t + NBUF < nt:
                    copy(t + NBUF).start()
                t += 1
            for j in range(NCHUNK):
                copy(t).wait()
                w = bufs[t % NBUF][:, pl.ds(0, H)]
                g_parts[j] = g_parts[j] + _dot_nt(h_prev, w)
                if t + NBUF < nt:
                    copy(t + NBUF).start()
                t += 1
            gates = (jnp.concatenate(g_parts, axis=1)
                     + bih_ref[pl.ds(idx, 1), :] + bhh_ref[pl.ds(idx, 1), :])
            i_ = jax.nn.sigmoid(gates[:, 0:H])
            f_ = jax.nn.sigmoid(gates[:, H:2 * H])
            g_ = jnp.tanh(gates[:, 2 * H:3 * H])
            o_ = jax.nn.sigmoid(gates[:, 3 * H:4 * H])
            c_new = f_ * c_prev + i_ * g_
            h_new = o_ * jnp.tanh(c_new)
            out_h_ref[pl.ds(idx, 1), :] = h_new
            out_c_ref[pl.ds(idx, 1), :] = c_new
            hs.append(h_new)
        x = jnp.concatenate(hs, axis=1)


def _lstm_tc(emb, h6, c6, bih6, bhh6, ws):
    vspec = pl.BlockSpec(memory_space=pltpu.MemorySpace.VMEM)
    aspec = pl.BlockSpec(memory_space=pl.ANY)
    return pl.pallas_call(
        _lstm_body,
        out_shape=[
            jax.ShapeDtypeStruct((2 * NL, H), jnp.float32),
            jax.ShapeDtypeStruct((2 * NL, H), jnp.float32),
        ],
        in_specs=[vspec] * 5 + [aspec] * 12,
        out_specs=[vspec, vspec],
        scratch_shapes=([pltpu.VMEM((R, 2 * H), jnp.float32)] * NBUF
                        + [pltpu.SemaphoreType.DMA] * NBUF),
    )(emb, h6, c6, bih6, bhh6, *ws)


def kernel(input, h0, c0, params):
    emb = _sc_gather(input, params["emb_table"])
    h6 = h0[:, 0, :]
    c6 = c0[:, 0, :]
    bih6 = jnp.stack([params[f"bih_{l}_{d}"] for l in range(NL) for d in range(2)])
    bhh6 = jnp.stack([params[f"bhh_{l}_{d}"] for l in range(NL) for d in range(2)])
    ws = []
    for l in range(NL):
        for d in range(2):
            ws.append(params[f"Wih_{l}_{d}"])
            ws.append(params[f"Whh_{l}_{d}"])
    out_h, out_c = _lstm_tc(emb, h6, c6, bih6, bhh6, ws)
    output = jnp.concatenate([out_h[2 * NL - 2], out_h[2 * NL - 1]], -1)[None, None, :]
    return (output, (out_h[:, None, :], out_c[:, None, :]))


# speedup vs baseline: 1.8568x; 1.8568x over previous
import functools

import jax
import jax.numpy as jnp
from jax import lax
from jax.experimental import pallas as pl
from jax.experimental.pallas import tpu as pltpu
from jax.experimental.pallas import tpu_sc as plsc

E = 128
H = 512
NL = 3
G = 4 * H
R = 512
NCHUNK = G // R
NBUF = 4


def _sc_gather(idx, table):
    _, e = table.shape
    mesh = plsc.VectorSubcoreMesh(core_axis_name="c", subcore_axis_name="s")

    @functools.partial(
        pl.kernel,
        out_type=jax.ShapeDtypeStruct((1, e), jnp.float32),
        mesh=mesh,
        scratch_types=[
            pltpu.VMEM((1,), jnp.int32),
            pltpu.VMEM((1, e), jnp.float32),
            pltpu.SemaphoreType.DMA,
        ],
    )
    def k(idx_hbm, table_hbm, out_hbm, idx_v, row_v, sem):
        c = lax.axis_index("c")
        s = lax.axis_index("s")

        @pl.when(jnp.logical_and(c == 0, s == 0))
        def _():
            pltpu.sync_copy(idx_hbm, idx_v)
            pltpu.async_copy(table_hbm.at[idx_v], row_v, sem).wait()
            pltpu.sync_copy(row_v, out_hbm)

    return k(idx, table)


def _matvec(w, x):
    r, k = w.shape
    acc = w[:, 0:128] * x[0:128]
    for g in range(1, k // 128):
        acc = acc + w[:, g * 128:(g + 1) * 128] * x[g * 128:(g + 1) * 128]
    return jnp.sum(acc, axis=1)


def _lstm_body(emb_ref, h6_ref, c6_ref, bih_ref, bhh_ref, *rest):
    w_refs = rest[:12]
    out_h_ref = rest[12]
    out_c_ref = rest[13]
    bufs = rest[14:14 + NBUF]
    sems = rest[14 + NBUF:14 + 2 * NBUF]

    tasks = []
    for l in range(NL):
        ind = E if l == 0 else 2 * H
        for d in range(2):
            m = 2 * (2 * l + d)
            for j in range(NCHUNK):
                tasks.append((m, j * R, ind))
            for j in range(NCHUNK):
                tasks.append((m + 1, j * R, H))
    nt = len(tasks)

    def copy(t):
        m, r0, cc = tasks[t]
        return pltpu.make_async_copy(
            w_refs[m].at[pl.ds(r0, R), pl.ds(0, cc)],
            bufs[t % NBUF].at[:, pl.ds(0, cc)],
            sems[t % NBUF],
        )

    for t in range(min(NBUF, nt)):
        copy(t).start()

    t = 0
    x = emb_ref[0, :]
    for l in range(NL):
        ind = E if l == 0 else 2 * H
        hs = []
        for d in range(2):
            idx = 2 * l + d
            h_prev = h6_ref[idx, :]
            c_prev = c6_ref[idx, :]
            g_parts = []
            for j in range(NCHUNK):
                copy(t).wait()
                w = bufs[t % NBUF][:, pl.ds(0, ind)]
                g_parts.append(_matvec(w, x))
                if t + NBUF < nt:
                    copy(t + NBUF).start()
                t += 1
            for j in range(NCHUNK):
                copy(t).wait()
                w = bufs[t % NBUF][:, pl.ds(0, H)]
                g_parts[j] = g_parts[j] + _matvec(w, h_prev)
                if t + NBUF < nt:
                    copy(t + NBUF).start()
                t += 1
            gates = (jnp.concatenate(g_parts, axis=0)
                     + bih_ref[idx, :] + bhh_ref[idx, :])
            i_ = jax.nn.sigmoid(gates[0:H])
            f_ = jax.nn.sigmoid(gates[H:2 * H])
            g_ = jnp.tanh(gates[2 * H:3 * H])
            o_ = jax.nn.sigmoid(gates[3 * H:4 * H])
            c_new = f_ * c_prev + i_ * g_
            h_new = o_ * jnp.tanh(c_new)
            out_h_ref[idx, :] = h_new
            out_c_ref[idx, :] = c_new
            hs.append(h_new)
        x = jnp.concatenate(hs, axis=0)


def _lstm_tc(emb, h6, c6, bih6, bhh6, ws):
    vspec = pl.BlockSpec(memory_space=pltpu.MemorySpace.VMEM)
    aspec = pl.BlockSpec(memory_space=pl.ANY)
    return pl.pallas_call(
        _lstm_body,
        out_shape=[
            jax.ShapeDtypeStruct((2 * NL, H), jnp.float32),
            jax.ShapeDtypeStruct((2 * NL, H), jnp.float32),
        ],
        in_specs=[vspec] * 5 + [aspec] * 12,
        out_specs=[vspec, vspec],
        scratch_shapes=([pltpu.VMEM((R, 2 * H), jnp.float32)] * NBUF
                        + [pltpu.SemaphoreType.DMA] * NBUF),
    )(emb, h6, c6, bih6, bhh6, *ws)


def kernel(input, h0, c0, params):
    emb = _sc_gather(input, params["emb_table"])
    h6 = h0[:, 0, :]
    c6 = c0[:, 0, :]
    bih6 = jnp.stack([params[f"bih_{l}_{d}"] for l in range(NL) for d in range(2)])
    bhh6 = jnp.stack([params[f"bhh_{l}_{d}"] for l in range(NL) for d in range(2)])
    ws = []
    for l in range(NL):
        for d in range(2):
            ws.append(params[f"Wih_{l}_{d}"])
            ws.append(params[f"Whh_{l}_{d}"])
    out_h, out_c = _lstm_tc(emb, h6, c6, bih6, bhh6, ws)
    output = jnp.concatenate([out_h[2 * NL - 2], out_h[2 * NL - 1]], -1)[None, None, :]
    return (output, (out_h[:, None, :], out_c[:, None, :]))


# speedup vs baseline: 1.9252x; 1.0368x over previous
import functools

import jax
import jax.numpy as jnp
from jax import lax
from jax.experimental import pallas as pl
from jax.experimental.pallas import tpu as pltpu
from jax.experimental.pallas import tpu_sc as plsc

E = 128
H = 512
NL = 3
G = 4 * H
R = 512
NCHUNK = G // R
NBUF = 8


def _sc_gather(idx, table):
    _, e = table.shape
    mesh = plsc.VectorSubcoreMesh(core_axis_name="c", subcore_axis_name="s")

    @functools.partial(
        pl.kernel,
        out_type=jax.ShapeDtypeStruct((1, e), jnp.float32),
        mesh=mesh,
        scratch_types=[
            pltpu.VMEM((1,), jnp.int32),
            pltpu.VMEM((1, e), jnp.float32),
            pltpu.SemaphoreType.DMA,
        ],
    )
    def k(idx_hbm, table_hbm, out_hbm, idx_v, row_v, sem):
        c = lax.axis_index("c")
        s = lax.axis_index("s")

        @pl.when(jnp.logical_and(c == 0, s == 0))
        def _():
            pltpu.sync_copy(idx_hbm, idx_v)
            pltpu.async_copy(table_hbm.at[idx_v], row_v, sem).wait()
            pltpu.sync_copy(row_v, out_hbm)

    return k(idx, table)


def _matvec_acc(w, x, acc):
    _, k = w.shape
    for g in range(k // 128):
        term = w[:, g * 128:(g + 1) * 128] * x[g * 128:(g + 1) * 128]
        acc = term if acc is None else acc + term
    return acc


def _lane_sum(acc):
    return jnp.sum(lax.transpose(acc, (1, 0)), axis=0)


def _lstm_body(emb_ref, h6_ref, c6_ref, bih_ref, bhh_ref, *rest):
    w_refs = rest[:12]
    out_h_ref = rest[12]
    out_c_ref = rest[13]
    bufs = rest[14:14 + NBUF]
    sems = rest[14 + NBUF:14 + 2 * NBUF]

    tasks = []
    for l in range(NL):
        ind = E if l == 0 else 2 * H
        for d in range(2):
            m = 2 * (2 * l + d)
            for j in range(NCHUNK):
                tasks.append((m, j * R, ind))
                tasks.append((m + 1, j * R, H))
    nt = len(tasks)

    def copy(t):
        m, r0, cc = tasks[t]
        return pltpu.make_async_copy(
            w_refs[m].at[pl.ds(r0, R), pl.ds(0, cc)],
            bufs[t % NBUF].at[:, pl.ds(0, cc)],
            sems[t % NBUF],
        )

    for t in range(min(NBUF, nt)):
        copy(t).start()

    t = 0
    x = emb_ref[0, :]
    for l in range(NL):
        ind = E if l == 0 else 2 * H
        hs = []
        for d in range(2):
            idx = 2 * l + d
            h_prev = h6_ref[idx, :]
            c_prev = c6_ref[idx, :]
            g_parts = []
            for j in range(NCHUNK):
                copy(t).wait()
                copy(t + 1).wait()
                bi = bufs[t % NBUF]
                bh = bufs[(t + 1) % NBUF]
                for rb in range(0, R, 128):
                    acc = _matvec_acc(bi[pl.ds(rb, 128), pl.ds(0, ind)], x, None)
                    acc = _matvec_acc(bh[pl.ds(rb, 128), pl.ds(0, H)], h_prev, acc)
                    g_parts.append(_lane_sum(acc))
                for tn in (t + NBUF, t + 1 + NBUF):
                    if tn < nt:
                        copy(tn).start()
                t += 2
            gates = (jnp.concatenate(g_parts, axis=0)
                     + bih_ref[idx, :] + bhh_ref[idx, :])
            i_ = jax.nn.sigmoid(gates[0:H])
            f_ = jax.nn.sigmoid(gates[H:2 * H])
            g_ = jnp.tanh(gates[2 * H:3 * H])
            o_ = jax.nn.sigmoid(gates[3 * H:4 * H])
            c_new = f_ * c_prev + i_ * g_
            h_new = o_ * jnp.tanh(c_new)
            out_h_ref[idx, :] = h_new
            out_c_ref[idx, :] = c_new
            hs.append(h_new)
        x = jnp.concatenate(hs, axis=0)


def _lstm_tc(emb, h6, c6, bih6, bhh6, ws):
    vspec = pl.BlockSpec(memory_space=pltpu.MemorySpace.VMEM)
    aspec = pl.BlockSpec(memory_space=pl.ANY)
    return pl.pallas_call(
        _lstm_body,
        out_shape=[
            jax.ShapeDtypeStruct((2 * NL, H), jnp.float32),
            jax.ShapeDtypeStruct((2 * NL, H), jnp.float32),
        ],
        in_specs=[vspec] * 5 + [aspec] * 12,
        out_specs=[vspec, vspec],
        scratch_shapes=([pltpu.VMEM((R, 2 * H), jnp.float32)] * NBUF
                        + [pltpu.SemaphoreType.DMA] * NBUF),
    )(emb, h6, c6, bih6, bhh6, *ws)


def kernel(input, h0, c0, params):
    emb = _sc_gather(input, params["emb_table"])
    h6 = h0[:, 0, :]
    c6 = c0[:, 0, :]
    bih6 = jnp.stack([params[f"bih_{l}_{d}"] for l in range(NL) for d in range(2)])
    bhh6 = jnp.stack([params[f"bhh_{l}_{d}"] for l in range(NL) for d in range(2)])
    ws = []
    for l in range(NL):
        for d in range(2):
            ws.append(params[f"Wih_{l}_{d}"])
            ws.append(params[f"Whh_{l}_{d}"])
    out_h, out_c = _lstm_tc(emb, h6, c6, bih6, bhh6, ws)
    output = jnp.concatenate([out_h[2 * NL - 2], out_h[2 * NL - 1]], -1)[None, None, :]
    return (output, (out_h[:, None, :], out_c[:, None, :]))
